# baseline (speedup 1.0000x reference)
import functools

import jax
import jax.numpy as jnp
from jax import lax
from jax.experimental import pallas as pl
from jax.experimental.pallas import tpu as pltpu
from jax.experimental.pallas import tpu_sc as plsc

N = 10000
E = 320000
D = 128
NC = 2
NS = 16
NW = NC * NS
EPW = E // NW
B = 80
NCH = EPW // B
RPT0 = 624
RPT_LAST = N - (NS - 1) * RPT0
CW = 128

_mesh = plsc.VectorSubcoreMesh(core_axis_name="c", subcore_axis_name="s")


def _zero_rows(ref, nrows, ncols):
    z = jnp.zeros((16,), jnp.float32)
    cpr = ncols // 16

    def body(i, carry):
        r = i // cpr
        c = (i % cpr) * 16
        ref[r, pl.ds(c, 16)] = z
        return carry

    lax.fori_loop(0, nrows * cpr, body, 0)


def _flush_zeros(zbuf, acc, row_base, nrows, bufrows):
    full = nrows // bufrows
    for k in range(full):
        pltpu.sync_copy(zbuf, acc.at[pl.ds(row_base + k * bufrows, bufrows)])
    rem = nrows - full * bufrows
    if rem:
        pltpu.sync_copy(zbuf.at[pl.ds(0, rem)],
                        acc.at[pl.ds(row_base + full * bufrows, rem)])


@functools.partial(
    pl.kernel,
    out_type=jax.ShapeDtypeStruct((NC * N, CW), jnp.float32),
    mesh=_mesh,
    scratch_types=[
        pltpu.VMEM_SHARED((N, CW), jnp.float32),
        pltpu.VMEM((B,), jnp.int32),
        pltpu.VMEM((B, CW), jnp.float32),
    ],
)
def _sc_counts(dst_hbm, out_hbm, acc, dst_v, ones_v):
    cid = lax.axis_index("c")
    sid = lax.axis_index("s")
    wid = cid * NS + sid
    row_base = sid * RPT0
    last = sid == NS - 1
    _zero_rows(ones_v, B, CW)

    @pl.when(jnp.logical_not(last))
    def _():
        _flush_zeros(ones_v, acc, row_base, RPT0, B)

    @pl.when(last)
    def _():
        _flush_zeros(ones_v, acc, row_base, RPT_LAST, B)

    e0 = jnp.where(lax.iota(jnp.int32, 16) == 0, 1.0, 0.0)

    def fill(i, carry):
        ones_v[i, pl.ds(0, 16)] = e0
        return carry

    lax.fori_loop(0, B, fill, 0)
    plsc.subcore_barrier()

    edge_base = wid * EPW

    def body(k, carry):
        base = edge_base + k * B
        pltpu.sync_copy(dst_hbm.at[pl.ds(base, B)], dst_v)
        pltpu.sync_copy(ones_v, acc.at[dst_v], add=True)
        return carry

    lax.fori_loop(0, NCH, body, 0)
    plsc.subcore_barrier()

    @pl.when(jnp.logical_not(last))
    def _():
        pltpu.sync_copy(acc.at[pl.ds(row_base, RPT0)],
                        out_hbm.at[pl.ds(cid * N + row_base, RPT0)])

    @pl.when(last)
    def _():
        pltpu.sync_copy(acc.at[pl.ds(row_base, RPT_LAST)],
                        out_hbm.at[pl.ds(cid * N + row_base, RPT_LAST)])


@functools.partial(
    pl.kernel,
    out_type=jax.ShapeDtypeStruct((NC * N, D), jnp.float32),
    mesh=_mesh,
    scratch_types=[
        pltpu.VMEM_SHARED((N, D), jnp.float32),
        pltpu.VMEM((EPW,), jnp.int32),
        pltpu.VMEM((EPW,), jnp.int32),
        pltpu.VMEM((3, B, D), jnp.float32),
        pltpu.SemaphoreType.DMA,
        pltpu.SemaphoreType.DMA,
        pltpu.SemaphoreType.DMA,
        pltpu.SemaphoreType.DMA,
        pltpu.SemaphoreType.DMA,
        pltpu.SemaphoreType.DMA,
        pltpu.SemaphoreType.DMA,
    ],
)
def _sc_scatter(h_hbm, src_hbm, dst_hbm, out_hbm, acc, src_v, dst_v, rows,
                semi, sg0, sg1, sg2, ss0, ss1, ss2):
    semg = [sg0, sg1, sg2]
    sems = [ss0, ss1, ss2]
    cid = lax.axis_index("c")
    sid = lax.axis_index("s")
    wid = cid * NS + sid
    row_base = sid * RPT0
    last = sid == NS - 1
    edge_base = wid * EPW
    cp_s = pltpu.async_copy(src_hbm.at[pl.ds(edge_base, EPW)], src_v, semi)
    cp_d = pltpu.async_copy(dst_hbm.at[pl.ds(edge_base, EPW)], dst_v, semi)
    _zero_rows(rows.at[0], B, D)

    @pl.when(jnp.logical_not(last))
    def _():
        _flush_zeros(rows.at[0], acc, row_base, RPT0, B)

    @pl.when(last)
    def _():
        _flush_zeros(rows.at[0], acc, row_base, RPT_LAST, B)

    plsc.subcore_barrier()
    cp_s.wait()
    cp_d.wait()

    def gather(k, b):
        pltpu.async_copy(h_hbm.at[src_v.at[pl.ds(k * B, B)]], rows.at[b],
                         semg[b])

    def wait_gather(k, b):
        pltpu.make_async_copy(h_hbm.at[src_v.at[pl.ds(k * B, B)]], rows.at[b],
                              semg[b]).wait()

    def scat(k, b):
        pltpu.async_copy(rows.at[b], acc.at[dst_v.at[pl.ds(k * B, B)]],
                         sems[b], add=True)

    def wait_scat(k, b):
        pltpu.make_async_copy(rows.at[b], acc.at[dst_v.at[pl.ds(k * B, B)]],
                              sems[b]).wait()

    for b in range(3):
        gather(b, b)

    def body(j, carry):
        k3 = 3 * j
        for b in range(3):
            wait_gather(k3 + b, b)
            scat(k3 + b, b)
        for b in range(3):
            wait_scat(k3 + b, b)

            @pl.when(k3 + b + 3 < NCH)
            def _():
                gather(k3 + b + 3, b)

        return carry

    lax.fori_loop(0, NCH // 3, body, 0)
    for k in range(3 * (NCH // 3), NCH):
        b = k % 3
        wait_gather(k, b)
        pltpu.sync_copy(rows.at[b], acc.at[dst_v.at[pl.ds(k * B, B)]],
                        add=True)
    plsc.subcore_barrier()

    @pl.when(jnp.logical_not(last))
    def _():
        pltpu.sync_copy(acc.at[pl.ds(row_base, RPT0)],
                        out_hbm.at[pl.ds(cid * N + row_base, RPT0)])

    @pl.when(last)
    def _():
        pltpu.sync_copy(acc.at[pl.ds(row_base, RPT_LAST)],
                        out_hbm.at[pl.ds(cid * N + row_base, RPT_LAST)])


_BR = 1000
_GRID = N // _BR


def _lin_body(x_ref, wt_ref, b_ref, o_ref):
    y = jnp.dot(x_ref[...], wt_ref[...], preferred_element_type=jnp.float32)
    o_ref[...] = jnp.maximum(y + b_ref[...], 0.0)


def _upd_body(h_ref, p0_ref, p1_ref, c0_ref, c1_ref, o_ref):
    cnt = c0_ref[:, 0:1] + c1_ref[:, 0:1]
    inv = 1.0 / jnp.maximum(cnt, 1.0)
    o_ref[...] = (h_ref[...] + (p0_ref[...] + p1_ref[...]) * inv) * 0.5


def _tail_body(h_ref, wpt_ref, wrt_ref, b_ref, o_ref):
    s = jnp.sum(jnp.maximum(h_ref[...], 0.0), axis=0, keepdims=True)
    g = jnp.dot(s * (1.0 / N), wpt_ref[...], preferred_element_type=jnp.float32)
    o_ref[...] = jnp.dot(g, wrt_ref[...], preferred_element_type=jnp.float32) \
        + b_ref[...]


def _tc_lin(x, wt, b):
    return pl.pallas_call(
        _lin_body,
        grid=(_GRID,),
        in_specs=[
            pl.BlockSpec((_BR, D), lambda i: (i, 0)),
            pl.BlockSpec((D, D), lambda i: (0, 0)),
            pl.BlockSpec((1, D), lambda i: (0, 0)),
        ],
        out_specs=pl.BlockSpec((_BR, D), lambda i: (i, 0)),
        out_shape=jax.ShapeDtypeStruct((N, D), jnp.float32),
    )(x, wt, b)


def _tc_update(h, parts, cparts):
    return pl.pallas_call(
        _upd_body,
        grid=(_GRID,),
        in_specs=[
            pl.BlockSpec((_BR, D), lambda i: (i, 0)),
            pl.BlockSpec((_BR, D), lambda i: (i, 0)),
            pl.BlockSpec((_BR, D), lambda i: (i + _GRID, 0)),
            pl.BlockSpec((_BR, CW), lambda i: (i, 0)),
            pl.BlockSpec((_BR, CW), lambda i: (i + _GRID, 0)),
        ],
        out_specs=pl.BlockSpec((_BR, D), lambda i: (i, 0)),
        out_shape=jax.ShapeDtypeStruct((N, D), jnp.float32),
    )(h, parts, parts, cparts, cparts)


def _tc_tail(h, wpt, wrt, b):
    return pl.pallas_call(
        _tail_body,
        grid=(1,),
        in_specs=[
            pl.BlockSpec((N, D), lambda i: (0, 0)),
            pl.BlockSpec((D, 64), lambda i: (0, 0)),
            pl.BlockSpec((64, 1), lambda i: (0, 0)),
            pl.BlockSpec((1, 1), lambda i: (0, 0)),
        ],
        out_specs=pl.BlockSpec((1, 1), lambda i: (0, 0)),
        out_shape=jax.ShapeDtypeStruct((1, 1), jnp.float32),
    )(h, wpt, wrt, b)


def kernel(x, edge_index, W_lin, b_lin, W_pool, W_read, b_read):
    src = edge_index[0].astype(jnp.int32)
    dst = edge_index[1].astype(jnp.int32)
    h = _tc_lin(x, W_lin.T, b_lin.reshape(1, D))
    cparts = _sc_counts(dst)
    for _ in range(3):
        parts = _sc_scatter(h, src, dst)
        h = _tc_update(h, parts, cparts)
    out = _tc_tail(h, W_pool.T, W_read.T, b_read.reshape(1, 1))
    return out.reshape(1)


# speedup vs baseline: 1.1067x; 1.1067x over previous
import functools

import jax
import jax.numpy as jnp
from jax import lax
from jax.experimental import pallas as pl
from jax.experimental.pallas import tpu as pltpu
from jax.experimental.pallas import tpu_sc as plsc

N = 10000
E = 320000
D = 128
NC = 2
NS = 16
NW = NC * NS
EPW = E // NW
B = 80
NCH = EPW // B
RPT0 = 624
RPT_LAST = N - (NS - 1) * RPT0
CW = 128

_mesh = plsc.VectorSubcoreMesh(core_axis_name="c", subcore_axis_name="s")


def _zero_rows(ref, nrows, ncols):
    z = jnp.zeros((16,), jnp.float32)
    cpr = ncols // 16

    def body(i, carry):
        r = i // cpr
        c = (i % cpr) * 16
        ref[r, pl.ds(c, 16)] = z
        return carry

    lax.fori_loop(0, nrows * cpr, body, 0)


def _flush_zeros(zbuf, acc, row_base, nrows, bufrows):
    full = nrows // bufrows
    for k in range(full):
        pltpu.sync_copy(zbuf, acc.at[pl.ds(row_base + k * bufrows, bufrows)])
    rem = nrows - full * bufrows
    if rem:
        pltpu.sync_copy(zbuf.at[pl.ds(0, rem)],
                        acc.at[pl.ds(row_base + full * bufrows, rem)])


@functools.partial(
    pl.kernel,
    out_type=jax.ShapeDtypeStruct((NC * N, CW), jnp.float32),
    mesh=_mesh,
    scratch_types=[
        pltpu.VMEM_SHARED((N, CW), jnp.float32),
        pltpu.VMEM((EPW,), jnp.int32),
        pltpu.VMEM((B, CW), jnp.float32),
        pltpu.SemaphoreType.DMA,
        pltpu.SemaphoreType.DMA,
        pltpu.SemaphoreType.DMA,
    ],
)
def _sc_counts(dst_hbm, out_hbm, acc, dst_v, ones_v, semi, ss0, ss1):
    cid = lax.axis_index("c")
    sid = lax.axis_index("s")
    wid = cid * NS + sid
    row_base = sid * RPT0
    last = sid == NS - 1
    cp_d = pltpu.async_copy(dst_hbm.at[pl.ds(wid * EPW, EPW)], dst_v, semi)
    _zero_rows(ones_v, B, CW)

    @pl.when(jnp.logical_not(last))
    def _():
        _flush_zeros(ones_v, acc, row_base, RPT0, B)

    @pl.when(last)
    def _():
        _flush_zeros(ones_v, acc, row_base, RPT_LAST, B)

    e0 = jnp.where(lax.iota(jnp.int32, 16) == 0, 1.0, 0.0)

    def fill(i, carry):
        ones_v[i, pl.ds(0, 16)] = e0
        return carry

    lax.fori_loop(0, B, fill, 0)
    plsc.subcore_barrier()
    cp_d.wait()

    def scat(k, sem):
        pltpu.async_copy(ones_v, acc.at[dst_v.at[pl.ds(k * B, B)]], sem,
                         add=True)

    def wait_scat(k, sem):
        pltpu.make_async_copy(ones_v, acc.at[dst_v.at[pl.ds(k * B, B)]],
                              sem).wait()

    scat(0, ss0)
    scat(1, ss1)

    def body(j, carry):
        k = 2 * j
        wait_scat(k, ss0)

        @pl.when(k + 2 < NCH)
        def _():
            scat(k + 2, ss0)

        wait_scat(k + 1, ss1)

        @pl.when(k + 3 < NCH)
        def _():
            scat(k + 3, ss1)

        return carry

    lax.fori_loop(0, (NCH - 1) // 2, body, 0)
    wait_scat(NCH - 1, ss0)
    plsc.subcore_barrier()

    @pl.when(jnp.logical_not(last))
    def _():
        pltpu.sync_copy(acc.at[pl.ds(row_base, RPT0)],
                        out_hbm.at[pl.ds(cid * N + row_base, RPT0)])

    @pl.when(last)
    def _():
        pltpu.sync_copy(acc.at[pl.ds(row_base, RPT_LAST)],
                        out_hbm.at[pl.ds(cid * N + row_base, RPT_LAST)])


@functools.partial(
    pl.kernel,
    out_type=jax.ShapeDtypeStruct((NC * N, D), jnp.float32),
    mesh=_mesh,
    scratch_types=[
        pltpu.VMEM_SHARED((N, D), jnp.float32),
        pltpu.VMEM((EPW,), jnp.int32),
        pltpu.VMEM((EPW,), jnp.int32),
        pltpu.VMEM((3, B, D), jnp.float32),
        pltpu.SemaphoreType.DMA,
        pltpu.SemaphoreType.DMA,
        pltpu.SemaphoreType.DMA,
        pltpu.SemaphoreType.DMA,
        pltpu.SemaphoreType.DMA,
        pltpu.SemaphoreType.DMA,
        pltpu.SemaphoreType.DMA,
    ],
)
def _sc_scatter(h_hbm, src_hbm, dst_hbm, out_hbm, acc, src_v, dst_v, rows,
                semi, sg0, sg1, sg2, ss0, ss1, ss2):
    semg = [sg0, sg1, sg2]
    sems = [ss0, ss1, ss2]
    cid = lax.axis_index("c")
    sid = lax.axis_index("s")
    wid = cid * NS + sid
    row_base = sid * RPT0
    last = sid == NS - 1
    edge_base = wid * EPW
    cp_s = pltpu.async_copy(src_hbm.at[pl.ds(edge_base, EPW)], src_v, semi)
    cp_d = pltpu.async_copy(dst_hbm.at[pl.ds(edge_base, EPW)], dst_v, semi)
    _zero_rows(rows.at[0], B, D)

    @pl.when(jnp.logical_not(last))
    def _():
        _flush_zeros(rows.at[0], acc, row_base, RPT0, B)

    @pl.when(last)
    def _():
        _flush_zeros(rows.at[0], acc, row_base, RPT_LAST, B)

    plsc.subcore_barrier()
    cp_s.wait()
    cp_d.wait()

    def gather(k, b):
        pltpu.async_copy(h_hbm.at[src_v.at[pl.ds(k * B, B)]], rows.at[b],
                         semg[b])

    def wait_gather(k, b):
        pltpu.make_async_copy(h_hbm.at[src_v.at[pl.ds(k * B, B)]], rows.at[b],
                              semg[b]).wait()

    def scat(k, b):
        pltpu.async_copy(rows.at[b], acc.at[dst_v.at[pl.ds(k * B, B)]],
                         sems[b], add=True)

    def wait_scat(k, b):
        pltpu.make_async_copy(rows.at[b], acc.at[dst_v.at[pl.ds(k * B, B)]],
                              sems[b]).wait()

    for b in range(3):
        gather(b, b)

    def body(j, carry):
        k3 = 3 * j
        for b in range(3):
            wait_gather(k3 + b, b)
            scat(k3 + b, b)
        for b in range(3):
            wait_scat(k3 + b, b)

            @pl.when(k3 + b + 3 < NCH)
            def _():
                gather(k3 + b + 3, b)

        return carry

    lax.fori_loop(0, NCH // 3, body, 0)
    for k in range(3 * (NCH // 3), NCH):
        b = k % 3
        wait_gather(k, b)
        pltpu.sync_copy(rows.at[b], acc.at[dst_v.at[pl.ds(k * B, B)]],
                        add=True)
    plsc.subcore_barrier()

    @pl.when(jnp.logical_not(last))
    def _():
        pltpu.sync_copy(acc.at[pl.ds(row_base, RPT0)],
                        out_hbm.at[pl.ds(cid * N + row_base, RPT0)])

    @pl.when(last)
    def _():
        pltpu.sync_copy(acc.at[pl.ds(row_base, RPT_LAST)],
                        out_hbm.at[pl.ds(cid * N + row_base, RPT_LAST)])


_BR = 1000
_GRID = N // _BR


def _lin_body(x_ref, wt_ref, b_ref, o_ref):
    y = jnp.dot(x_ref[...], wt_ref[...], preferred_element_type=jnp.float32)
    o_ref[...] = jnp.maximum(y + b_ref[...], 0.0)


def _upd_body(h_ref, p0_ref, p1_ref, c0_ref, c1_ref, o_ref):
    cnt = c0_ref[:, 0:1] + c1_ref[:, 0:1]
    inv = 1.0 / jnp.maximum(cnt, 1.0)
    o_ref[...] = (h_ref[...] + (p0_ref[...] + p1_ref[...]) * inv) * 0.5


def _tail_body(h_ref, wpt_ref, wrt_ref, b_ref, o_ref):
    s = jnp.sum(jnp.maximum(h_ref[...], 0.0), axis=0, keepdims=True)
    g = jnp.dot(s * (1.0 / N), wpt_ref[...], preferred_element_type=jnp.float32)
    o_ref[...] = jnp.dot(g, wrt_ref[...], preferred_element_type=jnp.float32) \
        + b_ref[...]


def _tc_lin(x, wt, b):
    return pl.pallas_call(
        _lin_body,
        grid=(_GRID,),
        in_specs=[
            pl.BlockSpec((_BR, D), lambda i: (i, 0)),
            pl.BlockSpec((D, D), lambda i: (0, 0)),
            pl.BlockSpec((1, D), lambda i: (0, 0)),
        ],
        out_specs=pl.BlockSpec((_BR, D), lambda i: (i, 0)),
        out_shape=jax.ShapeDtypeStruct((N, D), jnp.float32),
    )(x, wt, b)


def _tc_update(h, parts, c0, c1):
    return pl.pallas_call(
        _upd_body,
        grid=(_GRID,),
        in_specs=[
            pl.BlockSpec((_BR, D), lambda i: (i, 0)),
            pl.BlockSpec((_BR, D), lambda i: (i, 0)),
            pl.BlockSpec((_BR, D), lambda i: (i + _GRID, 0)),
            pl.BlockSpec((_BR, CW), lambda i: (i, 0)),
            pl.BlockSpec((_BR, CW), lambda i: (i + _GRID, 0)),
        ],
        out_specs=pl.BlockSpec((_BR, D), lambda i: (i, 0)),
        out_shape=jax.ShapeDtypeStruct((N, D), jnp.float32),
    )(h, parts, parts, c0, c1)


def _tc_tail(h, wpt, wrt, b):
    return pl.pallas_call(
        _tail_body,
        grid=(1,),
        in_specs=[
            pl.BlockSpec((N, D), lambda i: (0, 0)),
            pl.BlockSpec((D, 64), lambda i: (0, 0)),
            pl.BlockSpec((64, 1), lambda i: (0, 0)),
            pl.BlockSpec((1, 1), lambda i: (0, 0)),
        ],
        out_specs=pl.BlockSpec((1, 1), lambda i: (0, 0)),
        out_shape=jax.ShapeDtypeStruct((1, 1), jnp.float32),
    )(h, wpt, wrt, b)


def kernel(x, edge_index, W_lin, b_lin, W_pool, W_read, b_read):
    src = edge_index[0].astype(jnp.int32)
    dst = edge_index[1].astype(jnp.int32)
    h = _tc_lin(x, W_lin.T, b_lin.reshape(1, D))
    cparts = _sc_counts(dst)
    for _ in range(3):
        parts = _sc_scatter(h, src, dst)
        h = _tc_update(h, parts, cparts, cparts)
    out = _tc_tail(h, W_pool.T, W_read.T, b_read.reshape(1, 1))
    return out.reshape(1)


# speedup vs baseline: 1.1409x; 1.0309x over previous
import functools

import jax
import jax.numpy as jnp
from jax import lax
from jax.experimental import pallas as pl
from jax.experimental.pallas import tpu as pltpu
from jax.experimental.pallas import tpu_sc as plsc

N = 10000
E = 320000
D = 128
NC = 2
NS = 16
NW = NC * NS
EPW = E // NW
B = 80
NCH = EPW // B
RPT0 = 624
RPT_LAST = N - (NS - 1) * RPT0
CW = 128

_mesh = plsc.VectorSubcoreMesh(core_axis_name="c", subcore_axis_name="s")


def _zero_rows(ref, nrows, ncols):
    z = jnp.zeros((16,), jnp.float32)
    cpr = ncols // 16

    def body(i, carry):
        r = i // cpr
        c = (i % cpr) * 16
        ref[r, pl.ds(c, 16)] = z
        return carry

    lax.fori_loop(0, nrows * cpr, body, 0)


def _flush_zeros(zbuf, acc, row_base, nrows, bufrows):
    full = nrows // bufrows
    for k in range(full):
        pltpu.sync_copy(zbuf, acc.at[pl.ds(row_base + k * bufrows, bufrows)])
    rem = nrows - full * bufrows
    if rem:
        pltpu.sync_copy(zbuf.at[pl.ds(0, rem)],
                        acc.at[pl.ds(row_base + full * bufrows, rem)])


@functools.partial(
    pl.kernel,
    out_type=jax.ShapeDtypeStruct((NC * N, CW), jnp.float32),
    mesh=_mesh,
    scratch_types=[
        pltpu.VMEM_SHARED((N, CW), jnp.float32),
        pltpu.VMEM((EPW,), jnp.int32),
        pltpu.VMEM((B, CW), jnp.float32),
        pltpu.SemaphoreType.DMA,
        pltpu.SemaphoreType.DMA,
        pltpu.SemaphoreType.DMA,
    ],
)
def _sc_counts(dst_hbm, out_hbm, acc, dst_v, ones_v, semi, ss0, ss1):
    cid = lax.axis_index("c")
    sid = lax.axis_index("s")
    wid = cid * NS + sid
    row_base = sid * RPT0
    last = sid == NS - 1
    cp_d = pltpu.async_copy(dst_hbm.at[pl.ds(wid * EPW, EPW)], dst_v, semi)
    _zero_rows(ones_v, B, CW)

    @pl.when(jnp.logical_not(last))
    def _():
        _flush_zeros(ones_v, acc, row_base, RPT0, B)

    @pl.when(last)
    def _():
        _flush_zeros(ones_v, acc, row_base, RPT_LAST, B)

    e0 = jnp.where(lax.iota(jnp.int32, 16) == 0, 1.0, 0.0)

    def fill(i, carry):
        ones_v[i, pl.ds(0, 16)] = e0
        return carry

    lax.fori_loop(0, B, fill, 0)
    plsc.subcore_barrier()
    cp_d.wait()

    def scat(k, sem):
        pltpu.async_copy(ones_v, acc.at[dst_v.at[pl.ds(k * B, B)]], sem,
                         add=True)

    def wait_scat(k, sem):
        pltpu.make_async_copy(ones_v, acc.at[dst_v.at[pl.ds(k * B, B)]],
                              sem).wait()

    scat(0, ss0)
    scat(1, ss1)

    def body(j, carry):
        k = 2 * j
        wait_scat(k, ss0)

        @pl.when(k + 2 < NCH)
        def _():
            scat(k + 2, ss0)

        wait_scat(k + 1, ss1)

        @pl.when(k + 3 < NCH)
        def _():
            scat(k + 3, ss1)

        return carry

    lax.fori_loop(0, (NCH - 1) // 2, body, 0)
    wait_scat(NCH - 1, ss0)
    plsc.subcore_barrier()

    @pl.when(jnp.logical_not(last))
    def _():
        pltpu.sync_copy(acc.at[pl.ds(row_base, RPT0)],
                        out_hbm.at[pl.ds(cid * N + row_base, RPT0)])

    @pl.when(last)
    def _():
        pltpu.sync_copy(acc.at[pl.ds(row_base, RPT_LAST)],
                        out_hbm.at[pl.ds(cid * N + row_base, RPT_LAST)])


@functools.partial(
    pl.kernel,
    out_type=jax.ShapeDtypeStruct((NC * N, D), jnp.float32),
    mesh=_mesh,
    scratch_types=[
        pltpu.VMEM_SHARED((N, D), jnp.float32),
        pltpu.VMEM((EPW,), jnp.int32),
        pltpu.VMEM((EPW,), jnp.int32),
        pltpu.VMEM((3, B, D), jnp.float32),
        pltpu.SemaphoreType.DMA,
        pltpu.SemaphoreType.DMA,
        pltpu.SemaphoreType.DMA,
        pltpu.SemaphoreType.DMA,
        pltpu.SemaphoreType.DMA,
        pltpu.SemaphoreType.DMA,
        pltpu.SemaphoreType.DMA,
    ],
)
def _sc_scatter(h_hbm, src_hbm, dst_hbm, out_hbm, acc, src_v, dst_v, rows,
                semi, sg0, sg1, sg2, ss0, ss1, ss2):
    semg = [sg0, sg1, sg2]
    sems = [ss0, ss1, ss2]
    cid = lax.axis_index("c")
    sid = lax.axis_index("s")
    wid = cid * NS + sid
    row_base = sid * RPT0
    last = sid == NS - 1
    edge_base = wid * EPW
    cp_s = pltpu.async_copy(src_hbm.at[pl.ds(edge_base, EPW)], src_v, semi)
    cp_d = pltpu.async_copy(dst_hbm.at[pl.ds(edge_base, EPW)], dst_v, semi)
    _zero_rows(rows.at[0], B, D)

    @pl.when(jnp.logical_not(last))
    def _():
        _flush_zeros(rows.at[0], acc, row_base, RPT0, B)

    @pl.when(last)
    def _():
        _flush_zeros(rows.at[0], acc, row_base, RPT_LAST, B)

    plsc.subcore_barrier()
    cp_s.wait()
    cp_d.wait()

    def gather(k, b):
        pltpu.async_copy(h_hbm.at[src_v.at[pl.ds(k * B, B)]], rows.at[b],
                         semg[b])

    def wait_gather(k, b):
        pltpu.make_async_copy(h_hbm.at[src_v.at[pl.ds(k * B, B)]], rows.at[b],
                              semg[b]).wait()

    def scat(k, b):
        pltpu.async_copy(rows.at[b], acc.at[dst_v.at[pl.ds(k * B, B)]],
                         sems[b], add=True)

    def wait_scat(k, b):
        pltpu.make_async_copy(rows.at[b], acc.at[dst_v.at[pl.ds(k * B, B)]],
                              sems[b]).wait()

    for b in range(3):
        gather(b, b)

    def body(j, carry):
        k3 = 3 * j
        for b in range(3):
            wait_gather(k3 + b, b)
            scat(k3 + b, b)
        for b in range(3):
            wait_scat(k3 + b, b)

            @pl.when(k3 + b + 3 < NCH)
            def _():
                gather(k3 + b + 3, b)

        return carry

    lax.fori_loop(0, NCH // 3, body, 0)
    for k in range(3 * (NCH // 3), NCH):
        b = k % 3
        wait_gather(k, b)
        pltpu.sync_copy(rows.at[b], acc.at[dst_v.at[pl.ds(k * B, B)]],
                        add=True)
    plsc.subcore_barrier()

    @pl.when(jnp.logical_not(last))
    def _():
        pltpu.sync_copy(acc.at[pl.ds(row_base, RPT0)],
                        out_hbm.at[pl.ds(cid * N + row_base, RPT0)])

    @pl.when(last)
    def _():
        pltpu.sync_copy(acc.at[pl.ds(row_base, RPT_LAST)],
                        out_hbm.at[pl.ds(cid * N + row_base, RPT_LAST)])


_BR = 1000
_GRID = N // _BR


def _lin_body(x_ref, wt_ref, b_ref, o_ref):
    y = jnp.dot(x_ref[...], wt_ref[...], preferred_element_type=jnp.float32)
    o_ref[...] = jnp.maximum(y + b_ref[...], 0.0)


def _upd_body(h_ref, p0_ref, p1_ref, c0_ref, c1_ref, o_ref):
    cnt = c0_ref[:, 0:1] + c1_ref[:, 0:1]
    inv = 1.0 / jnp.maximum(cnt, 1.0)
    o_ref[...] = (h_ref[...] + (p0_ref[...] + p1_ref[...]) * inv) * 0.5


def _upd_tail_body(h_ref, p0_ref, p1_ref, c0_ref, c1_ref, wpt_ref, wrt_ref,
                   b_ref, o_ref, acc_ref):
    i = pl.program_id(0)
    cnt = c0_ref[:, 0:1] + c1_ref[:, 0:1]
    inv = 1.0 / jnp.maximum(cnt, 1.0)
    h3 = (h_ref[...] + (p0_ref[...] + p1_ref[...]) * inv) * 0.5
    part = jnp.sum(jnp.maximum(h3, 0.0), axis=0, keepdims=True)

    @pl.when(i == 0)
    def _():
        acc_ref[...] = jnp.zeros_like(acc_ref)

    acc_ref[...] += part

    @pl.when(i == _GRID - 1)
    def _():
        g = jnp.dot(acc_ref[...] * (1.0 / N), wpt_ref[...],
                    preferred_element_type=jnp.float32)
        o_ref[...] = jnp.dot(g, wrt_ref[...],
                             preferred_element_type=jnp.float32) + b_ref[...]


def _tail_body(h_ref, wpt_ref, wrt_ref, b_ref, o_ref):
    s = jnp.sum(jnp.maximum(h_ref[...], 0.0), axis=0, keepdims=True)
    g = jnp.dot(s * (1.0 / N), wpt_ref[...], preferred_element_type=jnp.float32)
    o_ref[...] = jnp.dot(g, wrt_ref[...], preferred_element_type=jnp.float32) \
        + b_ref[...]


def _tc_lin(x, wt, b):
    return pl.pallas_call(
        _lin_body,
        grid=(_GRID,),
        in_specs=[
            pl.BlockSpec((_BR, D), lambda i: (i, 0)),
            pl.BlockSpec((D, D), lambda i: (0, 0)),
            pl.BlockSpec((1, D), lambda i: (0, 0)),
        ],
        out_specs=pl.BlockSpec((_BR, D), lambda i: (i, 0)),
        out_shape=jax.ShapeDtypeStruct((N, D), jnp.float32),
    )(x, wt, b)


def _tc_update(h, parts, c0, c1):
    return pl.pallas_call(
        _upd_body,
        grid=(_GRID,),
        in_specs=[
            pl.BlockSpec((_BR, D), lambda i: (i, 0)),
            pl.BlockSpec((_BR, D), lambda i: (i, 0)),
            pl.BlockSpec((_BR, D), lambda i: (i + _GRID, 0)),
            pl.BlockSpec((_BR, 8), lambda i: (i, 0)),
            pl.BlockSpec((_BR, 8), lambda i: (i, 0)),
        ],
        out_specs=pl.BlockSpec((_BR, D), lambda i: (i, 0)),
        out_shape=jax.ShapeDtypeStruct((N, D), jnp.float32),
    )(h, parts, parts, c0, c1)


def _tc_upd_tail(h, parts, c0, c1, wpt, wrt, b):
    return pl.pallas_call(
        _upd_tail_body,
        grid=(_GRID,),
        in_specs=[
            pl.BlockSpec((_BR, D), lambda i: (i, 0)),
            pl.BlockSpec((_BR, D), lambda i: (i, 0)),
            pl.BlockSpec((_BR, D), lambda i: (i + _GRID, 0)),
            pl.BlockSpec((_BR, 8), lambda i: (i, 0)),
            pl.BlockSpec((_BR, 8), lambda i: (i, 0)),
            pl.BlockSpec((D, 64), lambda i: (0, 0)),
            pl.BlockSpec((64, 1), lambda i: (0, 0)),
            pl.BlockSpec((1, 1), lambda i: (0, 0)),
        ],
        out_specs=pl.BlockSpec((1, 1), lambda i: (0, 0)),
        out_shape=jax.ShapeDtypeStruct((1, 1), jnp.float32),
        scratch_shapes=[pltpu.VMEM((1, D), jnp.float32)],
    )(h, parts, parts, c0, c1, wpt, wrt, b)


def kernel(x, edge_index, W_lin, b_lin, W_pool, W_read, b_read):
    src = edge_index[0].astype(jnp.int32)
    dst = edge_index[1].astype(jnp.int32)
    h = _tc_lin(x, W_lin.T, b_lin.reshape(1, D))
    cparts = _sc_counts(dst)
    c0 = cparts[:N, :8]
    c1 = cparts[N:, :8]
    for _ in range(2):
        parts = _sc_scatter(h, src, dst)
        h = _tc_update(h, parts, c0, c1)
    parts = _sc_scatter(h, src, dst)
    out = _tc_upd_tail(h, parts, c0, c1, W_pool.T, W_read.T,
                       b_read.reshape(1, 1))
    return out.reshape(1)
